# baseline (speedup 1.0000x reference)
import jax
import jax.numpy as jnp
from jax import lax
from jax.experimental import pallas as pl
from jax.experimental.pallas import tpu as pltpu
from jax.experimental.pallas import tpu_sc as plsc

EMBED_DIM = 64
IN_DIM = 192
HIDDEN = 128
L = 16
SLAB = 128
B1_ROW = 192
W2_ROW = 193
B2_ROW = 194


_BCAST_DNUMS = lax.GatherDimensionNumbers(
    offset_dims=(), collapsed_slice_dims=(0,), start_index_map=(0,))


def _bcast_lane(ev, l):
    idx = jnp.full((L, 1), l, dtype=jnp.int32)
    return lax.gather(ev, idx, _BCAST_DNUMS, (1,),
                      mode=lax.GatherScatterMode.PROMISE_IN_BOUNDS)


def _xlane_sum(s):
    lane = lax.iota(jnp.int32, L)
    for sh in (8, 4, 2, 1):
        idx = ((lane + sh) & (L - 1)).reshape(L, 1)
        s = s + lax.gather(s, idx, _BCAST_DNUMS, (1,),
                           mode=lax.GatherScatterMode.PROMISE_IN_BOUNDS)
    return s


def _worker_id():
    return lax.axis_index("s") * 2 + lax.axis_index("c")


def _gather16(ref, rows, cols):
    return plsc.load_gather(ref, [rows, cols])


def _sc_body(idx_h, t0T_h, t1T_h, stT_h, w_h, out_h,
             idx_v, s0_v, s1_v, s2_v, w_v, out_v, sem_idx, sem_g, sem_w):
    wid = _worker_id()

    @pl.when(wid == 0)
    def _():
        idx_cp = pltpu.make_async_copy(idx_h, idx_v.at[pl.ds(0, 8)], sem_idx)
        idx_cp.start()
        w_cp = pltpu.make_async_copy(w_h, w_v, sem_w)
        w_cp.start()
        idx_cp.wait()

        iv = idx_v[...]
        bases = [pl.multiple_of((iv[r] >> 7) << 7, SLAB) for r in range(3)]
        g_cp = [
            pltpu.make_async_copy(stT_h.at[:, pl.ds(bases[0], SLAB)],
                                  s0_v, sem_g),
            pltpu.make_async_copy(t0T_h.at[:, pl.ds(bases[1], SLAB)],
                                  s1_v, sem_g),
            pltpu.make_async_copy(t1T_h.at[:, pl.ds(bases[2], SLAB)],
                                  s2_v, sem_g),
        ]
        for c in g_cp:
            c.start()
        col_all = iv & (SLAB - 1)
        cols = [_bcast_lane(col_all, r) for r in range(3)]
        w_cp.wait()
        for c in g_cp:
            c.wait()

        acc = [w_v[B1_ROW, pl.ds(16 * j, L)] for j in range(HIDDEN // L)]
        lane = lax.iota(jnp.int32, L)
        for r, slab_v in enumerate((s0_v, s1_v, s2_v)):
            evs = [_gather16(slab_v, lane + 16 * k, cols[r])
                   for k in range(EMBED_DIM // L)]
            ebs = [_bcast_lane(ev, l) for ev in evs for l in range(L)]
            for l in range(EMBED_DIM):
                d = r * EMBED_DIM + l
                for j in range(HIDDEN // L):
                    acc[j] = acc[j] + ebs[l] * w_v[d, pl.ds(16 * j, L)]

        s = jnp.zeros((L,), jnp.float32)
        for j in range(HIDDEN // L):
            h = jnp.maximum(acc[j], 0.0)
            s = s + h * w_v[W2_ROW, pl.ds(16 * j, L)]
        logit = _xlane_sum(s) + w_v[B2_ROW, pl.ds(0, L)]
        out_v[...] = 1.0 / (1.0 + jnp.exp(-logit))
        pltpu.sync_copy(out_v, out_h)


@jax.jit
def _run(idx_all, t0T, t1T, stT, wpack):
    mesh = plsc.VectorSubcoreMesh(core_axis_name="c", subcore_axis_name="s",
                                  num_cores=1, num_subcores=1)
    f = pl.kernel(
        _sc_body,
        out_type=jax.ShapeDtypeStruct((L,), jnp.float32),
        mesh=mesh,
        scratch_types=[
            pltpu.VMEM((L,), jnp.int32),
            pltpu.VMEM((EMBED_DIM, SLAB), jnp.float32),
            pltpu.VMEM((EMBED_DIM, SLAB), jnp.float32),
            pltpu.VMEM((EMBED_DIM, SLAB), jnp.float32),
            pltpu.VMEM((B2_ROW + 1, HIDDEN), jnp.float32),
            pltpu.VMEM((L,), jnp.float32),
            pltpu.SemaphoreType.DMA,
            pltpu.SemaphoreType.DMA,
            pltpu.SemaphoreType.DMA,
        ],
        compiler_params=pltpu.CompilerParams(needs_layout_passes=False,
                                             skip_device_barrier=True),
    )
    return f(idx_all, t0T, t1T, stT, wpack)


def kernel(speaker, word0, word1, table0, table1, speaker_table, W1, b1, W2, b2):
    idx_all = jnp.concatenate([
        speaker.astype(jnp.int32), word0.astype(jnp.int32),
        word1.astype(jnp.int32), jnp.zeros((5,), jnp.int32)])
    wpack = jnp.concatenate([
        W1, b1[None, :], W2.reshape(1, HIDDEN),
        jnp.pad(b2, (0, HIDDEN - 1))[None, :]], axis=0)
    res = _run(idx_all, table0.T, table1.T, speaker_table.T, wpack)
    return res[0:1].reshape(1, 1)


# speedup vs baseline: 1.2947x; 1.2947x over previous
import jax
import jax.numpy as jnp
from jax import lax
from jax.experimental import pallas as pl
from jax.experimental.pallas import tpu as pltpu
from jax.experimental.pallas import tpu_sc as plsc

EMBED_DIM = 64
IN_DIM = 192
HIDDEN = 128
L = 16
SLAB = 128
NT = 12
B1_ROW = 192
W2_ROW = 193
B2_ROW = 194
WPAD = 200


_BCAST_DNUMS = lax.GatherDimensionNumbers(
    offset_dims=(), collapsed_slice_dims=(0,), start_index_map=(0,))


def _bcast_lane(ev, l):
    idx = jnp.full((L, 1), l, dtype=jnp.int32)
    return lax.gather(ev, idx, _BCAST_DNUMS, (1,),
                      mode=lax.GatherScatterMode.PROMISE_IN_BOUNDS)


def _xlane_sum(s):
    lane = lax.iota(jnp.int32, L)
    for sh in (8, 4, 2, 1):
        idx = ((lane + sh) & (L - 1)).reshape(L, 1)
        s = s + lax.gather(s, idx, _BCAST_DNUMS, (1,),
                           mode=lax.GatherScatterMode.PROMISE_IN_BOUNDS)
    return s


def _gather16(ref, rows, cols):
    return plsc.load_gather(ref, [rows, cols])


def _sc_body(idx_h, t0T_h, t1T_h, stT_h, w_h, out_h,
             idx_v, slab_v, w16_v, part_v, acc_v, out_v, shared_v,
             sem_idx, sem_g, sem_w):
    wid = lax.axis_index("s")

    @pl.when(wid < NT)
    def _():
        idx_cp = pltpu.make_async_copy(idx_h, idx_v.at[pl.ds(0, 8)], sem_idx)
        idx_cp.start()
        woff = pl.multiple_of(wid * L, L)
        w_cp = pltpu.make_async_copy(w_h.at[pl.ds(woff, L)], w16_v, sem_w)
        w_cp.start()
        idx_cp.wait()

        iv = idx_v[...]
        r = wid >> 2
        roff = pl.multiple_of((wid & 3) * L, L)
        bases = [pl.multiple_of((iv[t] >> 7) << 7, SLAB) for t in range(3)]
        tabs = (stT_h, t0T_h, t1T_h)
        for rr in range(3):
            @pl.when(r == rr)
            def _():
                pltpu.make_async_copy(
                    tabs[rr].at[pl.ds(roff, L), pl.ds(bases[rr], SLAB)],
                    slab_v, sem_g).start()
        col = _bcast_lane(iv & (SLAB - 1), r)
        lane = lax.iota(jnp.int32, L)
        pltpu.make_async_copy(
            tabs[0].at[pl.ds(roff, L), pl.ds(bases[0], SLAB)],
            slab_v, sem_g).wait()
        w_cp.wait()

        ev = _gather16(slab_v, lane, col)
        acc = [jnp.zeros((L,), jnp.float32) for _ in range(HIDDEN // L)]
        for l in range(L):
            eb = _bcast_lane(ev, l)
            for j in range(HIDDEN // L):
                acc[j] = acc[j] + eb * w16_v[l, pl.ds(16 * j, L)]
        for j in range(HIDDEN // L):
            part_v[pl.ds(16 * j, L)] = acc[j]
        pltpu.sync_copy(part_v, shared_v.at[pl.ds(
            pl.multiple_of(HIDDEN * wid, HIDDEN), HIDDEN)])

    plsc.subcore_barrier()

    @pl.when(wid == 0)
    def _():
        tail_cp = pltpu.make_async_copy(
            w_h.at[pl.ds(B1_ROW, 8)], w16_v.at[pl.ds(0, 8)], sem_w)
        tail_cp.start()
        all_cp = pltpu.make_async_copy(shared_v, acc_v, sem_g)
        all_cp.start()
        tail_cp.wait()
        all_cp.wait()
        s = jnp.zeros((L,), jnp.float32)
        for j in range(HIDDEN // L):
            h = w16_v[0, pl.ds(16 * j, L)]
            for t in range(NT):
                h = h + acc_v[pl.ds(HIDDEN * t + 16 * j, L)]
            h = jnp.maximum(h, 0.0)
            s = s + h * w16_v[1, pl.ds(16 * j, L)]
        logit = _xlane_sum(s) + w16_v[2, pl.ds(0, L)]
        out_v[...] = 1.0 / (1.0 + jnp.exp(-logit))
        pltpu.sync_copy(out_v, out_h)


@jax.jit
def _run(idx_all, t0T, t1T, stT, wpack):
    mesh = plsc.VectorSubcoreMesh(core_axis_name="c", subcore_axis_name="s",
                                  num_cores=1, num_subcores=16)
    f = pl.kernel(
        _sc_body,
        out_type=jax.ShapeDtypeStruct((L,), jnp.float32),
        mesh=mesh,
        scratch_types=[
            pltpu.VMEM((L,), jnp.int32),
            pltpu.VMEM((L, SLAB), jnp.float32),
            pltpu.VMEM((L, HIDDEN), jnp.float32),
            pltpu.VMEM((HIDDEN,), jnp.float32),
            pltpu.VMEM((NT * HIDDEN,), jnp.float32),
            pltpu.VMEM((L,), jnp.float32),
            pltpu.VMEM_SHARED((NT * HIDDEN,), jnp.float32),
            pltpu.SemaphoreType.DMA,
            pltpu.SemaphoreType.DMA,
            pltpu.SemaphoreType.DMA,
        ],
        compiler_params=pltpu.CompilerParams(needs_layout_passes=False,
                                             skip_device_barrier=True),
    )
    return f(idx_all, t0T, t1T, stT, wpack)


def kernel(speaker, word0, word1, table0, table1, speaker_table, W1, b1, W2, b2):
    idx_all = jnp.concatenate([
        speaker.astype(jnp.int32), word0.astype(jnp.int32),
        word1.astype(jnp.int32), jnp.zeros((5,), jnp.int32)])
    wpack = jnp.concatenate([
        W1, b1[None, :], W2.reshape(1, HIDDEN),
        jnp.pad(b2, (0, HIDDEN - 1))[None, :],
        jnp.zeros((WPAD - B2_ROW - 1, HIDDEN), jnp.float32)], axis=0)
    res = _run(idx_all, table0.T, table1.T, speaker_table.T, wpack)
    return res[0:1].reshape(1, 1)


# speedup vs baseline: 1.3255x; 1.0238x over previous
import jax
import jax.numpy as jnp
from jax import lax
from jax.experimental import pallas as pl
from jax.experimental.pallas import tpu as pltpu
from jax.experimental.pallas import tpu_sc as plsc

EMBED_DIM = 64
IN_DIM = 192
HIDDEN = 128
L = 16
SLAB = 128
NT = 12
B1_ROW = 192
W2_ROW = 193
B2_ROW = 194
WPAD = 200


_BCAST_DNUMS = lax.GatherDimensionNumbers(
    offset_dims=(), collapsed_slice_dims=(0,), start_index_map=(0,))


def _bcast_lane(ev, l):
    idx = jnp.full((L, 1), l, dtype=jnp.int32)
    return lax.gather(ev, idx, _BCAST_DNUMS, (1,),
                      mode=lax.GatherScatterMode.PROMISE_IN_BOUNDS)


def _xlane_sum(s):
    lane = lax.iota(jnp.int32, L)
    for sh in (8, 4, 2, 1):
        idx = ((lane + sh) & (L - 1)).reshape(L, 1)
        s = s + lax.gather(s, idx, _BCAST_DNUMS, (1,),
                           mode=lax.GatherScatterMode.PROMISE_IN_BOUNDS)
    return s


def _gather16(ref, rows, cols):
    return plsc.load_gather(ref, [rows, cols])


def _sc_body(idx_h, t0T_h, t1T_h, stT_h, w_h, out_h,
             idx_v, slab_v, w16_v, wt_v, part_v, acc_v, out_v, shared_v,
             sem_idx, sem_g, sem_w):
    wid = lax.axis_index("s")

    @pl.when(wid < NT)
    def _():
        idx_cp = pltpu.make_async_copy(idx_h, idx_v.at[pl.ds(0, 8)], sem_idx)
        idx_cp.start()
        woff = pl.multiple_of(wid * L, L)
        w_cp = pltpu.make_async_copy(w_h.at[pl.ds(woff, L)], w16_v, sem_w)
        w_cp.start()
        idx_cp.wait()

        iv = idx_v[...]
        r = wid >> 2
        roff = pl.multiple_of((wid & 3) * L, L)
        bases = [pl.multiple_of((iv[t] >> 7) << 7, SLAB) for t in range(3)]
        tabs = (stT_h, t0T_h, t1T_h)
        for rr in range(3):
            @pl.when(r == rr)
            def _():
                pltpu.make_async_copy(
                    tabs[rr].at[pl.ds(roff, L), pl.ds(bases[rr], SLAB)],
                    slab_v, sem_g).start()
        col = _bcast_lane(iv & (SLAB - 1), r)
        lane = lax.iota(jnp.int32, L)
        pltpu.make_async_copy(
            tabs[0].at[pl.ds(roff, L), pl.ds(bases[0], SLAB)],
            slab_v, sem_g).wait()
        w_cp.wait()

        ev = _gather16(slab_v, lane, col)
        acc = [jnp.zeros((L,), jnp.float32) for _ in range(HIDDEN // L)]
        for l in range(L):
            eb = _bcast_lane(ev, l)
            for j in range(HIDDEN // L):
                acc[j] = acc[j] + eb * w16_v[l, pl.ds(16 * j, L)]
        for j in range(HIDDEN // L):
            part_v[pl.ds(16 * j, L)] = acc[j]
        pltpu.sync_copy(part_v, shared_v.at[pl.ds(
            pl.multiple_of(HIDDEN * wid, HIDDEN), HIDDEN)])

    @pl.when(wid == 0)
    def _():
        pltpu.make_async_copy(
            w_h.at[pl.ds(B1_ROW, 8)], wt_v, sem_w).start()

    plsc.subcore_barrier()

    @pl.when(wid == 0)
    def _():
        all_cp = pltpu.make_async_copy(shared_v, acc_v, sem_g)
        all_cp.start()
        pltpu.make_async_copy(
            w_h.at[pl.ds(B1_ROW, 8)], wt_v, sem_w).wait()
        all_cp.wait()
        s = jnp.zeros((L,), jnp.float32)
        for j in range(HIDDEN // L):
            h = wt_v[0, pl.ds(16 * j, L)]
            for t in range(NT):
                h = h + acc_v[pl.ds(HIDDEN * t + 16 * j, L)]
            h = jnp.maximum(h, 0.0)
            s = s + h * wt_v[1, pl.ds(16 * j, L)]
        logit = _xlane_sum(s) + wt_v[2, pl.ds(0, L)]
        out_v[...] = 1.0 / (1.0 + jnp.exp(-logit))
        pltpu.sync_copy(out_v, out_h)


@jax.jit
def _run(idx_all, t0T, t1T, stT, wpack):
    mesh = plsc.VectorSubcoreMesh(core_axis_name="c", subcore_axis_name="s",
                                  num_cores=1, num_subcores=16)
    f = pl.kernel(
        _sc_body,
        out_type=jax.ShapeDtypeStruct((L,), jnp.float32),
        mesh=mesh,
        scratch_types=[
            pltpu.VMEM((L,), jnp.int32),
            pltpu.VMEM((L, SLAB), jnp.float32),
            pltpu.VMEM((L, HIDDEN), jnp.float32),
            pltpu.VMEM((8, HIDDEN), jnp.float32),
            pltpu.VMEM((HIDDEN,), jnp.float32),
            pltpu.VMEM((NT * HIDDEN,), jnp.float32),
            pltpu.VMEM((L,), jnp.float32),
            pltpu.VMEM_SHARED((NT * HIDDEN,), jnp.float32),
            pltpu.SemaphoreType.DMA,
            pltpu.SemaphoreType.DMA,
            pltpu.SemaphoreType.DMA,
        ],
        compiler_params=pltpu.CompilerParams(needs_layout_passes=False,
                                             skip_device_barrier=True,
                                             disable_semaphore_checks=True),
    )
    return f(idx_all, t0T, t1T, stT, wpack)


def kernel(speaker, word0, word1, table0, table1, speaker_table, W1, b1, W2, b2):
    idx_all = jnp.concatenate([
        speaker.astype(jnp.int32), word0.astype(jnp.int32),
        word1.astype(jnp.int32), jnp.zeros((5,), jnp.int32)])
    wpack = jnp.concatenate([
        W1, b1[None, :], W2.reshape(1, HIDDEN),
        jnp.pad(b2, (0, HIDDEN - 1))[None, :],
        jnp.zeros((WPAD - B2_ROW - 1, HIDDEN), jnp.float32)], axis=0)
    res = _run(idx_all, table0.T, table1.T, speaker_table.T, wpack)
    return res[0:1].reshape(1, 1)
